# baseline (speedup 1.0000x reference)
import functools

import jax
import jax.numpy as jnp
from jax import lax
from jax.experimental import pallas as pl
from jax.experimental.pallas import tpu as pltpu
from jax.experimental.pallas import tpu_sc as plsc


def kernel(output_of_intra_node_moe_tensor, x, route_prob_max, indexes):
    batch, seq, d_model = x.shape
    tokens = batch * seq
    src = output_of_intra_node_moe_tensor
    idx_flat = indexes.reshape(-1).astype(jnp.int32)
    prob = route_prob_max

    info = plsc.get_sparse_core_info()
    num_workers = info.num_cores * info.num_subcores
    lanes = info.num_lanes
    rows_per_worker = tokens // num_workers
    chunk = lanes
    n_chunks = rows_per_worker // chunk
    half = rows_per_worker // 2

    mesh = plsc.VectorSubcoreMesh(core_axis_name="c", subcore_axis_name="s")

    @functools.partial(
        pl.kernel,
        mesh=mesh,
        out_type=jax.ShapeDtypeStruct((tokens, d_model), jnp.float32),
        scratch_types=[
            pltpu.VMEM((rows_per_worker,), jnp.int32),
            pltpu.VMEM((2, half), jnp.int32),
            pltpu.VMEM((rows_per_worker,), jnp.float32),
            pltpu.VMEM((chunk, d_model), jnp.float32),
            pltpu.VMEM((chunk, d_model), jnp.float32),
            pltpu.VMEM((chunk, d_model), jnp.float32),
            pltpu.SemaphoreType.DMA,
            pltpu.SemaphoreType.DMA,
            pltpu.SemaphoreType.DMA,
            pltpu.SemaphoreType.DMA,
            pltpu.SemaphoreType.DMA,
            pltpu.SemaphoreType.DMA,
            pltpu.SemaphoreType.DMA,
        ],
    )
    def sc_kernel(src_hbm, idx_hbm, prob_hbm, out_hbm, idx_v, idx2, probs_v,
                  rows0, rows1, rows2, psem, ld0, ld1, ld2, st0, st1, st2):
        wid = lax.axis_index("s") * info.num_cores + lax.axis_index("c")
        base = wid * rows_per_worker
        pltpu.sync_copy(idx_hbm.at[pl.ds(base, rows_per_worker)], idx_v)
        pltpu.sync_copy(idx_hbm.at[pl.ds(base, half)], idx2.at[0])
        pltpu.sync_copy(idx_hbm.at[pl.ds(base + half, half)], idx2.at[1])

        rows = (rows0, rows1, rows2)
        lds = (ld0, ld1, ld2)
        sts = (st0, st1, st2)

        pltpu.async_copy(prob_hbm.at[idx2.at[0]], probs_v.at[pl.ds(0, half)],
                         psem)
        pltpu.async_copy(prob_hbm.at[idx2.at[1]],
                         probs_v.at[pl.ds(half, half)], psem)

        def issue_load(c, k):
            pltpu.async_copy(
                src_hbm.at[pl.ds(base + c * chunk, chunk)], rows[k], lds[k])

        def wait_load(k):
            pltpu.make_async_copy(
                src_hbm.at[pl.ds(0, chunk)], rows[k], lds[k]).wait()

        def wait_scatter(k):
            pltpu.make_async_copy(
                rows[k], out_hbm.at[idx_v[pl.ds(0, chunk)]], sts[k]).wait()

        dnums = lax.GatherDimensionNumbers(
            offset_dims=(), collapsed_slice_dims=(0,), start_index_map=(0,))

        def compute(c, k):
            pv = probs_v[pl.ds(c * chunk, chunk)]
            buf = rows[k]
            for j in range(chunk):
                pj = lax.gather(
                    pv, jnp.full((lanes, 1), j, jnp.int32), dnums,
                    slice_sizes=(1,),
                    mode=lax.GatherScatterMode.PROMISE_IN_BOUNDS)

                @plsc.parallel_loop(0, d_model // lanes, unroll=16)
                def _row(i, j=j, pj=pj, buf=buf):
                    sl = pl.ds(i * lanes, lanes)
                    buf[j, sl] = buf[j, sl] * pj

        issue_load(0, 0)
        issue_load(1, 1)
        pltpu.make_async_copy(
            prob_hbm.at[idx2.at[0]], probs_v.at[pl.ds(0, half)], psem).wait()
        pltpu.make_async_copy(
            prob_hbm.at[idx2.at[1]], probs_v.at[pl.ds(half, half)],
            psem).wait()

        def do_chunk(c, k):
            k2 = (k + 2) % 3
            wait_load(k)

            @pl.when(c > 0)
            def _():
                wait_scatter(k2)

            @pl.when(c < n_chunks - 2)
            def _():
                issue_load(c + 2, k2)

            compute(c, k)
            dst = idx_v[pl.ds(c * chunk, chunk)]
            pltpu.async_copy(rows[k], out_hbm.at[dst], sts[k])

        def group_body(g, carry):
            c = 3 * g
            do_chunk(c, 0)
            do_chunk(c + 1, 1)
            do_chunk(c + 2, 2)
            return carry

        lax.fori_loop(0, n_chunks // 3, group_body, 0)
        do_chunk(n_chunks - 1, (n_chunks - 1) % 3)
        wait_scatter((n_chunks - 1) % 3)

    out = sc_kernel(src, idx_flat, prob)
    return out.reshape(batch, seq, d_model)


# speedup vs baseline: 1.0797x; 1.0797x over previous
import functools

import jax
import jax.numpy as jnp
from jax import lax
from jax.experimental import pallas as pl
from jax.experimental.pallas import tpu as pltpu
from jax.experimental.pallas import tpu_sc as plsc


def kernel(output_of_intra_node_moe_tensor, x, route_prob_max, indexes):
    batch, seq, d_model = x.shape
    tokens = batch * seq
    src = output_of_intra_node_moe_tensor
    idx_flat = indexes.reshape(-1).astype(jnp.int32)
    prob = route_prob_max

    info = plsc.get_sparse_core_info()
    num_workers = info.num_cores * info.num_subcores
    lanes = info.num_lanes
    rows_per_worker = tokens // num_workers
    chunk = 8
    n_chunks = rows_per_worker // chunk
    half = rows_per_worker // 2
    idx2d = idx_flat.reshape(tokens // chunk, chunk)

    mesh = plsc.VectorSubcoreMesh(core_axis_name="c", subcore_axis_name="s")

    @functools.partial(
        pl.kernel,
        mesh=mesh,
        out_type=jax.ShapeDtypeStruct((tokens, d_model), jnp.float32),
        scratch_types=[
            pltpu.VMEM((n_chunks, chunk), jnp.int32),
            pltpu.VMEM((2, half), jnp.int32),
            pltpu.VMEM((rows_per_worker,), jnp.float32),
            pltpu.VMEM((chunk, d_model), jnp.float32),
            pltpu.VMEM((chunk, d_model), jnp.float32),
            pltpu.VMEM((chunk, d_model), jnp.float32),
            pltpu.VMEM((chunk, d_model), jnp.float32),
            pltpu.SemaphoreType.DMA,
            pltpu.SemaphoreType.DMA,
            pltpu.SemaphoreType.DMA,
            pltpu.SemaphoreType.DMA,
            pltpu.SemaphoreType.DMA,
        ],
    )
    def sc_kernel(src_hbm, idx_hbm, idx2d_hbm, prob_hbm, out_hbm, idx8, idx2,
                  probs_v, lbuf0, lbuf1, sbuf0, sbuf1, psem, ld0, ld1, st0,
                  st1):
        wid = lax.axis_index("s") * info.num_cores + lax.axis_index("c")
        base = wid * rows_per_worker
        pltpu.sync_copy(idx2d_hbm.at[pl.ds(wid * n_chunks, n_chunks)], idx8)
        pltpu.sync_copy(idx_hbm.at[pl.ds(base, half)], idx2.at[0])
        pltpu.sync_copy(idx_hbm.at[pl.ds(base + half, half)], idx2.at[1])

        lbufs = (lbuf0, lbuf1)
        sbufs = (sbuf0, sbuf1)
        lds = (ld0, ld1)
        sts = (st0, st1)

        pltpu.async_copy(prob_hbm.at[idx2.at[0]], probs_v.at[pl.ds(0, half)],
                         psem)
        pltpu.async_copy(prob_hbm.at[idx2.at[1]],
                         probs_v.at[pl.ds(half, half)], psem)

        def issue_load(c, q):
            pltpu.async_copy(
                src_hbm.at[pl.ds(base + c * chunk, chunk)], lbufs[q], lds[q])

        def wait_load(q):
            pltpu.make_async_copy(
                src_hbm.at[pl.ds(0, chunk)], lbufs[q], lds[q]).wait()

        def wait_scatter(q):
            pltpu.make_async_copy(
                sbufs[q], out_hbm.at[idx8.at[0]], sts[q]).wait()

        dnums = lax.GatherDimensionNumbers(
            offset_dims=(), collapsed_slice_dims=(0,), start_index_map=(0,))

        def compute(pv16, q):
            for j in range(chunk):
                pj = lax.gather(
                    pv16, jnp.full((lanes, 1), q * chunk + j, jnp.int32),
                    dnums, slice_sizes=(1,),
                    mode=lax.GatherScatterMode.PROMISE_IN_BOUNDS)

                @plsc.parallel_loop(0, d_model // lanes, unroll=8)
                def _row(i, j=j, pj=pj, q=q):
                    sl = pl.ds(i * lanes, lanes)
                    sbufs[q][j, sl] = lbufs[q][j, sl] * pj

        issue_load(0, 0)
        issue_load(1, 1)
        pltpu.make_async_copy(
            prob_hbm.at[idx2.at[0]], probs_v.at[pl.ds(0, half)], psem).wait()
        pltpu.make_async_copy(
            prob_hbm.at[idx2.at[1]], probs_v.at[pl.ds(half, half)],
            psem).wait()

        def do_chunk(c, q, pv16):
            wait_load(q)

            @pl.when(c >= 2)
            def _():
                wait_scatter(q)

            compute(pv16, q)

            @pl.when(c < n_chunks - 2)
            def _():
                issue_load(c + 2, q)

            pltpu.async_copy(sbufs[q], out_hbm.at[idx8.at[c]], sts[q])

        def pair_body(g, carry):
            pv16 = probs_v[pl.ds(g * lanes, lanes)]
            do_chunk(2 * g, 0, pv16)
            do_chunk(2 * g + 1, 1, pv16)
            return carry

        lax.fori_loop(0, n_chunks // 2, pair_body, 0)
        wait_scatter(0)
        wait_scatter(1)

    out = sc_kernel(src, idx_flat, idx2d, prob)
    return out.reshape(batch, seq, d_model)
